# Initial kernel scaffold; baseline (speedup 1.0000x reference)
import jax
import jax.numpy as jnp
from jax.experimental import pallas as pl


def kernel(x, edge_index, W0, b0, W1, b1, W2, b2, W3, b3):
    raise NotImplementedError("write your pallas kernel here")


# speedup vs baseline: 3.3490x; 3.3490x over previous
import dataclasses
import functools

import jax
import jax.numpy as jnp
from jax import lax
from jax.experimental import pallas as pl
from jax.experimental.pallas import tpu as pltpu
from jax.experimental.pallas import tpu_sc as plsc

N = 10000
E = 160000
D = 256
LANES = 16
NC = 2
NS = 16
NW = NC * NS
NPT = 320
NPAD = NW * NPT
CE = 2000
GC = 64
DC = D // LANES

_mesh = plsc.VectorSubcoreMesh(core_axis_name="c", subcore_axis_name="s")

_sc_params = pltpu.CompilerParams()
if "needs_layout_passes" in pltpu.CompilerParams.__dataclass_fields__:
    _sc_params = dataclasses.replace(_sc_params, needs_layout_passes=False)


def _wid():
    return lax.axis_index("c") * NS + lax.axis_index("s")


def _prep_body(src_hbm, dst_hbm, srcb_hbm, dstb_hbm, cntb_hbm, deg_hbm,
               dstc, srcc, sbuf, dbuf, deg2, stg):
    wid = _wid()
    iota = lax.iota(jnp.int32, LANES)
    zi = jnp.zeros((LANES,), jnp.int32)
    ones = jnp.ones((LANES,), jnp.int32)

    @pl.loop(0, NPT)
    def _(r):
        deg2[pl.ds(r * LANES, LANES)] = zi

    @pl.loop(0, sbuf.shape[0] // LANES)
    def _(r):
        sbuf[pl.ds(r * LANES, LANES)] = zi
        dbuf[pl.ds(r * LANES, LANES)] = zi

    def chunk(ci, carry):
        pos0, off0 = carry
        pltpu.sync_copy(dst_hbm.at[pl.ds(ci * CE, CE)], dstc)
        pltpu.sync_copy(src_hbm.at[pl.ds(ci * CE, CE)], srcc)

        def group(g, pos):
            d16 = dstc[pl.ds(g * LANES, LANES)]
            s16 = srcc[pl.ds(g * LANES, LANES)]
            t16 = d16 // NPT
            m = t16 == wid
            dl16 = d16 - wid * NPT
            plsc.store_compressed(sbuf.at[pl.ds(pos, LANES)], s16, mask=m)
            plsc.store_compressed(dbuf.at[pl.ds(pos, LANES)], dl16, mask=m)
            plsc.addupdate_scatter(deg2, [dl16 * LANES + iota], ones, mask=m)
            return pos + plsc.all_reduce_population_count(m)[0]

        pos = lax.fori_loop(0, CE // LANES, group, pos0)

        nflush = pos // GC

        @pl.loop(0, nflush)
        def _(f):
            o = pl.multiple_of(wid * E + off0 + f * GC, GC)
            pltpu.sync_copy(sbuf.at[pl.ds(f * GC, GC)], srcb_hbm.at[pl.ds(o, GC)])
            pltpu.sync_copy(dbuf.at[pl.ds(f * GC, GC)], dstb_hbm.at[pl.ds(o, GC)])

        base = nflush * GC
        for q in range(GC // LANES):
            sbuf[pl.ds(q * LANES, LANES)] = sbuf[pl.ds(base + q * LANES, LANES)]
            dbuf[pl.ds(q * LANES, LANES)] = dbuf[pl.ds(base + q * LANES, LANES)]
        return pos - base, off0 + base

    pos, off = lax.fori_loop(0, E // CE, chunk, (jnp.int32(0), jnp.int32(0)))

    nfin = (pos + GC - 1) // GC

    @pl.loop(0, nfin)
    def _(f):
        o = pl.multiple_of(wid * E + off + f * GC, GC)
        pltpu.sync_copy(sbuf.at[pl.ds(f * GC, GC)], srcb_hbm.at[pl.ds(o, GC)])
        pltpu.sync_copy(dbuf.at[pl.ds(f * GC, GC)], dstb_hbm.at[pl.ds(o, GC)])

    cnt = off + pos
    stg[pl.ds(0, LANES)] = jnp.full((LANES,), cnt, jnp.int32)
    pltpu.sync_copy(stg, cntb_hbm.at[pl.ds(pl.multiple_of(wid * LANES, LANES), LANES)])
    pltpu.sync_copy(deg2, deg_hbm.at[pl.ds(pl.multiple_of(wid * NPT * LANES, LANES), NPT * LANES)])


@jax.jit
def _prep(src, dst):
    k = pl.kernel(
        _prep_body,
        out_type=(
            jax.ShapeDtypeStruct((NW * E,), jnp.int32),
            jax.ShapeDtypeStruct((NW * E,), jnp.int32),
            jax.ShapeDtypeStruct((NW * LANES,), jnp.int32),
            jax.ShapeDtypeStruct((NW * NPT * LANES,), jnp.int32),
        ),
        mesh=_mesh,
        scratch_types=[
            pltpu.VMEM((CE,), jnp.int32),
            pltpu.VMEM((CE,), jnp.int32),
            pltpu.VMEM((CE + 2 * GC,), jnp.int32),
            pltpu.VMEM((CE + 2 * GC,), jnp.int32),
            pltpu.VMEM((NPT * LANES,), jnp.int32),
            pltpu.VMEM((LANES,), jnp.int32),
        ],
        compiler_params=_sc_params,
    )
    return k(src, dst)


def _scale_body(x_ref, deg_ref, xs_ref, dinv_ref):
    deg = jnp.sum(deg_ref[...], axis=1, keepdims=True).astype(jnp.float32)
    dinv = lax.rsqrt(deg + 1.0)
    dinv_ref[...] = dinv
    xs_ref[...] = x_ref[...] * dinv


@jax.jit
def _scale(x_pad, deg16):
    bm = 1024
    return pl.pallas_call(
        _scale_body,
        grid=(NPAD // bm,),
        in_specs=[
            pl.BlockSpec((bm, D), lambda i: (i, 0)),
            pl.BlockSpec((bm, LANES), lambda i: (i, 0)),
        ],
        out_specs=[
            pl.BlockSpec((bm, D), lambda i: (i, 0)),
            pl.BlockSpec((bm, 1), lambda i: (i, 0)),
        ],
        out_shape=[
            jax.ShapeDtypeStruct((NPAD, D), jnp.float32),
            jax.ShapeDtypeStruct((NPAD, 1), jnp.float32),
        ],
    )(x_pad, deg16)


def _mm_body(x_ref, w_ref, o_ref):
    o_ref[...] = jnp.dot(x_ref[...], w_ref[...],
                         preferred_element_type=jnp.float32)


@jax.jit
def _mm(x, w):
    bm = 1024
    return pl.pallas_call(
        _mm_body,
        grid=(NPAD // bm,),
        in_specs=[
            pl.BlockSpec((bm, D), lambda i: (i, 0)),
            pl.BlockSpec((D, D), lambda i: (0, 0)),
        ],
        out_specs=pl.BlockSpec((bm, D), lambda i: (i, 0)),
        out_shape=jax.ShapeDtypeStruct((NPAD, D), jnp.float32),
    )(x, w)


def _agg_body(prescale, h_hbm, srcb_hbm, dstb_hbm, cntb_hbm, dinv_hbm, b_hbm,
              o_hbm, acc, rbuf, ibuf, dlc, dinvb, bb, stg):
    wid = _wid()
    iota = lax.iota(jnp.int32, LANES)
    zf = jnp.zeros((LANES,), jnp.float32)

    pltpu.sync_copy(cntb_hbm.at[pl.ds(pl.multiple_of(wid * LANES, LANES), LANES)], stg)
    cnt = stg[pl.ds(0, LANES)][0]
    pltpu.sync_copy(dinv_hbm.at[pl.ds(pl.multiple_of(wid * NPT, NPT), NPT)], dinvb)
    pltpu.sync_copy(b_hbm, bb)

    @pl.loop(0, NPT)
    def _(r):
        for c in range(DC):
            acc[r, pl.ds(c * LANES, LANES)] = zf

    nch = (cnt + GC - 1) // GC

    @pl.loop(0, nch)
    def _(i):
        o = pl.multiple_of(wid * E + i * GC, GC)
        pltpu.sync_copy(srcb_hbm.at[pl.ds(o, GC)], ibuf)
        pltpu.sync_copy(dstb_hbm.at[pl.ds(o, GC)], dlc)
        pltpu.sync_copy(h_hbm.at[ibuf], rbuf)

        @pl.loop(0, GC // LANES)
        def _(q):
            dl16 = dlc[pl.ds(q * LANES, LANES)]
            base = i * GC + q * LANES
            for k in range(LANES):
                dl = dl16[k]
                row = jnp.full((LANES,), dl, jnp.int32)
                m = jnp.full((LANES,), base + k < cnt)
                for c in range(DC):
                    plsc.addupdate_scatter(
                        acc, [row, iota + c * LANES],
                        rbuf[q * LANES + k, pl.ds(c * LANES, LANES)], mask=m)

    @pl.loop(0, NPT // GC)
    def _(blk):
        pltpu.sync_copy(
            h_hbm.at[pl.ds(pl.multiple_of(wid * NPT + blk * GC, GC), GC)], rbuf)

        @pl.loop(0, GC // LANES)
        def _(rg):
            dv16 = dinvb[pl.ds(blk * GC + rg * LANES, LANES)]
            for k in range(LANES):
                s = dv16[k]
                f1 = s * s if prescale else s
                r = blk * GC + rg * LANES + k
                for c in range(DC):
                    v = acc[r, pl.ds(c * LANES, LANES)]
                    hv = rbuf[rg * LANES + k, pl.ds(c * LANES, LANES)]
                    bv = bb[pl.ds(c * LANES, LANES)]
                    y = f1 * (v + hv) + (s * bv if prescale else bv)
                    acc[r, pl.ds(c * LANES, LANES)] = y

    pltpu.sync_copy(acc, o_hbm.at[pl.ds(wid * NPT, NPT)])


def _make_agg(prescale):
    return pl.kernel(
        functools.partial(_agg_body, prescale),
        out_type=jax.ShapeDtypeStruct((NPAD, D), jnp.float32),
        mesh=_mesh,
        scratch_types=[
            pltpu.VMEM((NPT, D), jnp.float32),
            pltpu.VMEM((GC, D), jnp.float32),
            pltpu.VMEM((GC,), jnp.int32),
            pltpu.VMEM((GC,), jnp.int32),
            pltpu.VMEM((NPT,), jnp.float32),
            pltpu.VMEM((D,), jnp.float32),
            pltpu.VMEM((LANES,), jnp.int32),
        ],
        compiler_params=_sc_params,
    )


_agg_mid = jax.jit(_make_agg(True))
_agg_last = jax.jit(_make_agg(False))


def kernel(x, edge_index, W0, b0, W1, b1, W2, b2, W3, b3):
    x_pad = jnp.pad(x, ((0, NPAD - N), (0, 0)))
    srcb, dstb, cntb, deg2 = _prep(edge_index[0], edge_index[1])
    deg16 = deg2.reshape(NPAD, LANES)
    xs, dinv_col = _scale(x_pad, deg16)
    dinv = dinv_col.reshape(NPAD)
    cur = xs
    for i, (W, b) in enumerate([(W0, b0), (W1, b1), (W2, b2), (W3, b3)]):
        h = _mm(cur, W)
        agg = _agg_mid if i < 3 else _agg_last
        cur = agg(h, srcb, dstb, cntb, dinv, b)
    return cur[:N]
